# Initial kernel scaffold; baseline (speedup 1.0000x reference)
import functools

import jax
import jax.numpy as jnp
from jax import lax
from jax.experimental import pallas as pl
from jax.experimental.pallas import tpu as pltpu
from jax.experimental.pallas import tpu_sc as plsc

D = 64
CH = 128
IDXBUF = 1024


@functools.lru_cache(maxsize=None)
def _make(B, F):
    info = plsc.get_sparse_core_info()
    NC, NS = info.num_cores, info.num_subcores
    NW = NC * NS
    R = B * F
    rows_u = B // NW
    rows_f = R // NW

    mesh = plsc.VectorSubcoreMesh(core_axis_name="c", subcore_axis_name="s")

    @functools.partial(
        pl.kernel,
        mesh=mesh,
        out_type=(
            jax.ShapeDtypeStruct((B, D), jnp.float32),
            jax.ShapeDtypeStruct((R, D), jnp.float32),
        ),
        scratch_types=[
            pltpu.VMEM((IDXBUF,), jnp.int32),
            pltpu.VMEM((CH, D), jnp.float32),
            pltpu.SemaphoreType.DMA,
        ],
    )
    def k(users_hbm, feats_hbm, utab, itab, uout, fout, idx_v, rows_v, sem):
        wid = lax.axis_index("s") * NC + lax.axis_index("c")

        def run(idx_hbm, tab, out, base, total):
            n_outer = total // IDXBUF if total >= IDXBUF else 1
            chunk = IDXBUF if total >= IDXBUF else total

            def outer(i, _):
                ib = pl.multiple_of(base + i * chunk, 8)
                pltpu.sync_copy(idx_hbm.at[pl.ds(ib, chunk)],
                                idx_v.at[pl.ds(0, chunk)])

                def inner(j, _):
                    pltpu.async_copy(
                        tab.at[idx_v.at[pl.ds(j * CH, CH)]], rows_v, sem
                    ).wait()
                    ob = pl.multiple_of(ib + j * CH, 8)
                    pltpu.sync_copy(rows_v, out.at[pl.ds(ob, CH)])
                    return 0

                lax.fori_loop(0, chunk // CH, inner, 0)
                return 0

            lax.fori_loop(0, n_outer, outer, 0)

        run(users_hbm, utab, uout, wid * rows_u, rows_u)
        run(feats_hbm, itab, fout, wid * rows_f, rows_f)

    return k


def kernel(users, feats, user_table, item_table):
    B = users.shape[0]
    F = feats.shape[1]
    k = _make(B, F)
    uout, fout = k(users, feats.reshape(-1), user_table, item_table)
    return (uout, fout.reshape(B, F, D))


# speedup vs baseline: 3.1654x; 3.1654x over previous
import functools

import jax
import jax.numpy as jnp
from jax import lax
from jax.experimental import pallas as pl
from jax.experimental.pallas import tpu as pltpu
from jax.experimental.pallas import tpu_sc as plsc

D = 64
CH = 128
IDXBUF = 1024


@functools.lru_cache(maxsize=None)
def _make(B, F):
    info = plsc.get_sparse_core_info()
    NC, NS = info.num_cores, info.num_subcores
    NW = NC * NS
    R = B * F
    rows_u = B // NW
    rows_f = R // NW

    mesh = plsc.VectorSubcoreMesh(core_axis_name="c", subcore_axis_name="s")

    @functools.partial(
        pl.kernel,
        mesh=mesh,
        out_type=(
            jax.ShapeDtypeStruct((B, D), jnp.float32),
            jax.ShapeDtypeStruct((R, D), jnp.float32),
        ),
        scratch_types=[
            pltpu.VMEM((IDXBUF,), jnp.int32),
            pltpu.VMEM((CH, D), jnp.float32),
            pltpu.SemaphoreType.DMA,
        ],
        compiler_params=pltpu.CompilerParams(use_tc_tiling_on_sc=False),
    )
    def k(users_hbm, feats_hbm, utab, itab, uout, fout, idx_v, rows_v, sem):
        wid = lax.axis_index("s") * NC + lax.axis_index("c")

        def run(idx_hbm, tab, out, base, total):
            n_outer = total // IDXBUF if total >= IDXBUF else 1
            chunk = IDXBUF if total >= IDXBUF else total

            def outer(i, _):
                ib = pl.multiple_of(base + i * chunk, 8)
                pltpu.sync_copy(idx_hbm.at[pl.ds(ib, chunk)],
                                idx_v.at[pl.ds(0, chunk)])

                def inner(j, _):
                    pltpu.async_copy(
                        tab.at[idx_v.at[pl.ds(j * CH, CH)]], rows_v, sem
                    ).wait()
                    ob = pl.multiple_of(ib + j * CH, 8)
                    pltpu.sync_copy(rows_v, out.at[pl.ds(ob, CH)])
                    return 0

                lax.fori_loop(0, chunk // CH, inner, 0)
                return 0

            lax.fori_loop(0, n_outer, outer, 0)

        run(users_hbm, utab, uout, wid * rows_u, rows_u)
        run(feats_hbm, itab, fout, wid * rows_f, rows_f)

    return k


def kernel(users, feats, user_table, item_table):
    B = users.shape[0]
    F = feats.shape[1]
    k = _make(B, F)
    uout, fout = k(users, feats.reshape(-1), user_table, item_table)
    return (uout, fout.reshape(B, F, D))


# speedup vs baseline: 3.5617x; 1.1252x over previous
import functools

import jax
import jax.numpy as jnp
from jax import lax
from jax.experimental import pallas as pl
from jax.experimental.pallas import tpu as pltpu
from jax.experimental.pallas import tpu_sc as plsc

D = 64
CH = 128
NBUF = 8
K = 6


@functools.lru_cache(maxsize=None)
def _make(B, F):
    info = plsc.get_sparse_core_info()
    NC, NS = info.num_cores, info.num_subcores
    NW = NC * NS
    R = B * F
    rows_u = B // NW
    rows_f = R // NW

    mesh = plsc.VectorSubcoreMesh(core_axis_name="c", subcore_axis_name="s")

    @functools.partial(
        pl.kernel,
        mesh=mesh,
        out_type=(
            jax.ShapeDtypeStruct((B, D), jnp.float32),
            jax.ShapeDtypeStruct((R, D), jnp.float32),
        ),
        scratch_types=[
            pltpu.VMEM((rows_u,), jnp.int32),
            pltpu.VMEM((rows_f,), jnp.int32),
            pltpu.VMEM((NBUF * CH, D), jnp.float32),
            pltpu.SemaphoreType.DMA,
            pltpu.SemaphoreType.DMA,
            pltpu.SemaphoreType.DMA,
        ],
        compiler_params=pltpu.CompilerParams(use_tc_tiling_on_sc=False),
    )
    def k(users_hbm, feats_hbm, utab, itab, uout, fout,
          uidx, fidx, rows, gsem, ssem, isem):
        wid = lax.axis_index("s") * NC + lax.axis_index("c")

        def buf(g):
            return rows.at[pl.ds((g % NBUF) * CH, CH)]

        def pipe(idx_v, tab, out, out_base, n_chunks):
            def fire_gather(g):
                pltpu.async_copy(tab.at[idx_v.at[pl.ds(g * CH, CH)]],
                                 buf(g), gsem)

            def wait_gather():
                pltpu.make_async_copy(tab.at[pl.ds(0, CH)], buf(0),
                                      gsem).wait()

            def fire_store(g):
                ob = pl.multiple_of(out_base + g * CH, 8)
                pltpu.async_copy(buf(g), out.at[pl.ds(ob, CH)], ssem)

            def drain_store():
                pltpu.make_async_copy(out.at[pl.ds(out_base, CH)],
                                      rows.at[pl.ds(0, CH)], ssem).wait()

            if n_chunks <= K:
                for g in range(n_chunks):
                    fire_gather(g)
                for g in range(n_chunks):
                    wait_gather()
                    fire_store(g)
                for g in range(n_chunks):
                    drain_store()
                return

            for g in range(K):
                fire_gather(g)
            for g in range(2):
                wait_gather()
                fire_store(g)
                fire_gather(g + K)

            def body(g, _):
                wait_gather()
                fire_store(g)
                drain_store()
                fire_gather(g + K)
                return 0

            lax.fori_loop(2, n_chunks - K, body, 0)

            for g in range(n_chunks - K, n_chunks):
                wait_gather()
                fire_store(g)
                drain_store()
            drain_store()
            drain_store()

        fcopy = pltpu.async_copy(
            feats_hbm.at[pl.ds(wid * rows_f, rows_f)], fidx, isem)
        pltpu.sync_copy(users_hbm.at[pl.ds(wid * rows_u, rows_u)], uidx)
        pipe(uidx, utab, uout, wid * rows_u, rows_u // CH)
        fcopy.wait()
        pipe(fidx, itab, fout, wid * rows_f, rows_f // CH)

    return k


def kernel(users, feats, user_table, item_table):
    B = users.shape[0]
    F = feats.shape[1]
    k = _make(B, F)
    uout, fout = k(users, feats.reshape(-1), user_table, item_table)
    return (uout, fout.reshape(B, F, D))
